# baseline (speedup 1.0000x reference)
import functools

import jax
import jax.numpy as jnp
from jax.experimental import pallas as pl

_S = 64.0
_M = 0.4

_BLOCK_ROWS = 16


def _cosface_block(cosine_ref, label_ref, out_ref):
    cols = jax.lax.broadcasted_iota(
        jnp.int32, (_BLOCK_ROWS, cosine_ref.shape[1]), 1)
    lbl = label_ref[...]
    margin = jnp.where(cols == lbl, -_S * _M, 0.0).astype(cosine_ref.dtype)
    out_ref[...] = cosine_ref[...] * _S + margin


@functools.partial(jax.jit, static_argnames=())
def kernel(cosine, label):
    rows, n_cols = cosine.shape
    grid = (pl.cdiv(rows, _BLOCK_ROWS),)
    lbl2d = label.reshape(rows, 1)
    return pl.pallas_call(
        _cosface_block,
        grid=grid,
        in_specs=[
            pl.BlockSpec((_BLOCK_ROWS, n_cols), lambda i: (i, 0)),
            pl.BlockSpec((_BLOCK_ROWS, 1), lambda i: (i, 0)),
        ],
        out_specs=pl.BlockSpec((_BLOCK_ROWS, n_cols), lambda i: (i, 0)),
        out_shape=jax.ShapeDtypeStruct((rows, n_cols), cosine.dtype),
    )(cosine, lbl2d)
